# baseline (speedup 1.0000x reference)
import jax
import jax.numpy as jnp
from jax import lax
from jax.experimental import pallas as pl
from jax.experimental.pallas import tpu as pltpu
from jax.experimental.pallas import tpu_sc as plsc

_V, _E, _B, _L = 100000, 100, 4096, 200
_EP = 128
_NC, _NS = 2, 16
_NW = _NC * _NS
_BPW = _B // _NW
_IPC = 100
_CPW = _BPW * _L // _IPC
_NBUF = 4


def _pool_body(x_hbm, table_hbm, out_hbm, xbuf, g0, g1, g2, g3,
               accbuf, s0, s1, s2, s3):
    gb = (g0, g1, g2, g3)
    sm = (s0, s1, s2, s3)
    wid = lax.axis_index("s") * _NC + lax.axis_index("c")
    pltpu.sync_copy(x_hbm.at[pl.ds(wid * _CPW, _CPW)], xbuf)

    for p in range(_NBUF):
        pltpu.async_copy(table_hbm.at[xbuf.at[p]], gb[p], sm[p])

    zero = jnp.zeros((16,), jnp.float32)

    def accum(g, acc):
        def r_body(r, acc):
            return tuple(acc[s] + g[r, pl.ds(s * 16, 16)] for s in range(8))
        return lax.fori_loop(0, _IPC, r_body, acc)

    def process(c, fire_next):
        for e in range(2):
            acc = (zero,) * 8
            for h in range(2):
                p = 2 * e + h
                chunk = c + p
                pltpu.make_async_copy(
                    table_hbm.at[xbuf.at[chunk]], gb[p], sm[p]).wait()
                acc = accum(gb[p], acc)
                if fire_next:
                    pltpu.async_copy(
                        table_hbm.at[xbuf.at[chunk + _NBUF]], gb[p], sm[p])
            elem = (c // 2) + e
            for s in range(8):
                accbuf[elem, pl.ds(s * 16, 16)] = acc[s]

    @pl.loop(0, _CPW - _NBUF, step=_NBUF)
    def _main(c):
        process(c, True)

    process(_CPW - _NBUF, False)

    pltpu.sync_copy(accbuf, out_hbm.at[pl.ds(wid * _BPW, _BPW)])


def _sc_pool(x2, table_p):
    mesh = plsc.VectorSubcoreMesh(
        core_axis_name="c", subcore_axis_name="s",
        num_cores=_NC, num_subcores=_NS)
    return pl.kernel(
        _pool_body,
        out_type=jax.ShapeDtypeStruct((_B, _EP), jnp.float32),
        mesh=mesh,
        compiler_params=pltpu.CompilerParams(use_tc_tiling_on_sc=True),
        scratch_types=[
            pltpu.VMEM((_CPW, _IPC), jnp.int32),
            pltpu.VMEM((_IPC, _EP), jnp.float32),
            pltpu.VMEM((_IPC, _EP), jnp.float32),
            pltpu.VMEM((_IPC, _EP), jnp.float32),
            pltpu.VMEM((_IPC, _EP), jnp.float32),
            pltpu.VMEM((_BPW, _EP), jnp.float32),
            pltpu.SemaphoreType.DMA,
            pltpu.SemaphoreType.DMA,
            pltpu.SemaphoreType.DMA,
            pltpu.SemaphoreType.DMA,
        ],
    )(x2, table_p)


_TPB = 1024


def _tp_body(t_ref, o_ref):
    blk = t_ref[...]
    eye = jnp.eye(_E, dtype=jnp.float32)
    bt = jax.lax.dot_general(blk, eye, (((0,), (0,)), ((), ())),
                             preferred_element_type=jnp.float32)
    o_ref[...] = jnp.concatenate(
        [bt, jnp.zeros((_TPB, _EP - _E), jnp.float32)], axis=1)


def _tc_pad_transpose(t100):
    return pl.pallas_call(
        _tp_body,
        grid=(pl.cdiv(_V, _TPB),),
        in_specs=[pl.BlockSpec((_E, _TPB), lambda i: (0, i))],
        out_specs=pl.BlockSpec((_TPB, _EP), lambda i: (i, 0)),
        out_shape=jax.ShapeDtypeStruct((_V, _EP), jnp.float32),
    )(t100)


def _mlp_body(p_ref, w1_ref, b1_ref, w2_ref, b2_ref, w3_ref, b3_ref, o_ref):
    h = jnp.dot(p_ref[...], w1_ref[...].T, preferred_element_type=jnp.float32)
    h = jnp.maximum(h + b1_ref[...], 0.0)
    h = jnp.dot(h, w2_ref[...].T, preferred_element_type=jnp.float32)
    h = jnp.maximum(h + b2_ref[...], 0.0)
    o_ref[...] = (
        jnp.dot(h, w3_ref[...].T, preferred_element_type=jnp.float32)
        + b3_ref[...])


def _tc_mlp(pooled, W1p, b1, W2, b2, W3, b3):
    return pl.pallas_call(
        _mlp_body,
        out_shape=jax.ShapeDtypeStruct((_B, W3.shape[0]), jnp.float32),
    )(pooled, W1p, b1.reshape(1, -1), W2, b2.reshape(1, -1),
      W3, b3.reshape(1, -1))


@jax.jit
def kernel(x, table, W1, b1, W2, b2, W3, b3):
    x2 = x.reshape(_B * _L // _IPC, _IPC)
    table_p = _tc_pad_transpose(table.T)
    W1p = jnp.pad(W1, ((0, 0), (0, _EP - _E)))
    pooled = _sc_pool(x2, table_p)
    return _tc_mlp(pooled, W1p, b1, W2, b2, W3, b3)


# speedup vs baseline: 1.1956x; 1.1956x over previous
import jax
import jax.numpy as jnp
from jax import lax
from jax.experimental import pallas as pl
from jax.experimental.pallas import tpu as pltpu
from jax.experimental.pallas import tpu_sc as plsc

_V, _E, _B, _L = 100000, 100, 4096, 200
_EP = 128
_NC, _NS = 2, 16
_NW = _NC * _NS
_BPW = _B // _NW
_IPC = 100
_CPW = _BPW * _L // _IPC
_NBUF = 4


def _pool_body(x_hbm, table_hbm, out_hbm, xbuf, g0, g1, g2, g3,
               accbuf, s0, s1, s2, s3):
    gb = (g0, g1, g2, g3)
    sm = (s0, s1, s2, s3)
    wid = lax.axis_index("s") * _NC + lax.axis_index("c")
    pltpu.sync_copy(x_hbm.at[pl.ds(wid * _CPW, _CPW)], xbuf)

    for p in range(_NBUF):
        pltpu.async_copy(table_hbm.at[xbuf.at[p]], gb[p], sm[p])

    zero = jnp.zeros((16,), jnp.float32)

    def accum(g, acc):
        def r_body(r, acc):
            return tuple(acc[s] + g[r, pl.ds(s * 16, 16)] for s in range(8))
        return lax.fori_loop(0, _IPC, r_body, acc)

    def process(c, fire_next):
        for e in range(2):
            acc = (zero,) * 8
            for h in range(2):
                p = 2 * e + h
                chunk = c + p
                pltpu.make_async_copy(
                    table_hbm.at[xbuf.at[chunk]], gb[p], sm[p]).wait()
                acc = accum(gb[p], acc)
                if fire_next:
                    pltpu.async_copy(
                        table_hbm.at[xbuf.at[chunk + _NBUF]], gb[p], sm[p])
            elem = (c // 2) + e
            for s in range(8):
                accbuf[elem, pl.ds(s * 16, 16)] = acc[s]

    @pl.loop(0, _CPW - _NBUF, step=_NBUF)
    def _main(c):
        process(c, True)

    process(_CPW - _NBUF, False)

    pltpu.sync_copy(accbuf, out_hbm.at[pl.ds(wid * _BPW, _BPW)])


def _sc_pool(x2, table_p):
    mesh = plsc.VectorSubcoreMesh(
        core_axis_name="c", subcore_axis_name="s",
        num_cores=_NC, num_subcores=_NS)
    return pl.kernel(
        _pool_body,
        out_type=jax.ShapeDtypeStruct((_B, _EP), jnp.float32),
        mesh=mesh,
        compiler_params=pltpu.CompilerParams(use_tc_tiling_on_sc=True),
        scratch_types=[
            pltpu.VMEM((_CPW, _IPC), jnp.int32),
            pltpu.VMEM((_IPC, _EP), jnp.float32),
            pltpu.VMEM((_IPC, _EP), jnp.float32),
            pltpu.VMEM((_IPC, _EP), jnp.float32),
            pltpu.VMEM((_IPC, _EP), jnp.float32),
            pltpu.VMEM((_BPW, _EP), jnp.float32),
            pltpu.SemaphoreType.DMA,
            pltpu.SemaphoreType.DMA,
            pltpu.SemaphoreType.DMA,
            pltpu.SemaphoreType.DMA,
        ],
    )(x2, table_p)


_TPB = 4096


def _tp_body(t_ref, o_ref):
    blk = t_ref[...]
    o_ref[...] = jnp.concatenate(
        [blk.T, jnp.zeros((_TPB, _EP - _E), jnp.float32)], axis=1)


def _tc_pad_transpose(t100):
    return pl.pallas_call(
        _tp_body,
        grid=(pl.cdiv(_V, _TPB),),
        in_specs=[pl.BlockSpec((_E, _TPB), lambda i: (0, i))],
        out_specs=pl.BlockSpec((_TPB, _EP), lambda i: (i, 0)),
        out_shape=jax.ShapeDtypeStruct((_V, _EP), jnp.float32),
    )(t100)


def _mlp_body(p_ref, w1_ref, b1_ref, w2_ref, b2_ref, w3_ref, b3_ref, o_ref):
    h = jnp.dot(p_ref[...], w1_ref[...].T, preferred_element_type=jnp.float32)
    h = jnp.maximum(h + b1_ref[...], 0.0)
    h = jnp.dot(h, w2_ref[...].T, preferred_element_type=jnp.float32)
    h = jnp.maximum(h + b2_ref[...], 0.0)
    o_ref[...] = (
        jnp.dot(h, w3_ref[...].T, preferred_element_type=jnp.float32)
        + b3_ref[...])


def _tc_mlp(pooled, W1p, b1, W2, b2, W3, b3):
    return pl.pallas_call(
        _mlp_body,
        out_shape=jax.ShapeDtypeStruct((_B, W3.shape[0]), jnp.float32),
    )(pooled, W1p, b1.reshape(1, -1), W2, b2.reshape(1, -1),
      W3, b3.reshape(1, -1))


@jax.jit
def kernel(x, table, W1, b1, W2, b2, W3, b3):
    x2 = x.reshape(_B * _L // _IPC, _IPC)
    table_p = _tc_pad_transpose(table.T)
    W1p = jnp.pad(W1, ((0, 0), (0, _EP - _E)))
    pooled = _sc_pool(x2, table_p)
    return _tc_mlp(pooled, W1p, b1, W2, b2, W3, b3)


# speedup vs baseline: 1.2360x; 1.0338x over previous
import jax
import jax.numpy as jnp
from jax import lax
from jax.experimental import pallas as pl
from jax.experimental.pallas import tpu as pltpu
from jax.experimental.pallas import tpu_sc as plsc

_V, _E, _B, _L = 100000, 100, 4096, 200
_EP = 128
_NC, _NS = 2, 16
_NW = _NC * _NS
_BPW = _B // _NW
_IPC = 100
_CPW = _BPW * _L // _IPC
_NBUF = 4


def _pool_body(x_hbm, table_hbm, out_hbm, xbuf, g0, g1, g2, g3,
               accbuf, s0, s1, s2, s3):
    gb = (g0, g1, g2, g3)
    sm = (s0, s1, s2, s3)
    wid = lax.axis_index("s") * _NC + lax.axis_index("c")
    pltpu.sync_copy(x_hbm.at[pl.ds(wid * _CPW, _CPW)], xbuf)

    for p in range(_NBUF):
        pltpu.async_copy(table_hbm.at[xbuf.at[p]], gb[p], sm[p])

    zero = jnp.zeros((16,), jnp.float32)

    def accum(g, acc):
        def r_body(r, acc):
            return tuple(acc[s] + g[r, pl.ds(s * 16, 16)] for s in range(8))
        return lax.fori_loop(0, _IPC, r_body, acc)

    def process(c, fire_next):
        for e in range(2):
            acc = (zero,) * 8
            for h in range(2):
                p = 2 * e + h
                chunk = c + p
                pltpu.make_async_copy(
                    table_hbm.at[xbuf.at[chunk]], gb[p], sm[p]).wait()
                acc = accum(gb[p], acc)
                if fire_next:
                    pltpu.async_copy(
                        table_hbm.at[xbuf.at[chunk + _NBUF]], gb[p], sm[p])
            elem = (c // 2) + e
            for s in range(8):
                accbuf[elem, pl.ds(s * 16, 16)] = acc[s]

    @pl.loop(0, _CPW - _NBUF, step=_NBUF)
    def _main(c):
        process(c, True)

    process(_CPW - _NBUF, False)

    pltpu.sync_copy(accbuf, out_hbm.at[pl.ds(wid * _BPW, _BPW)])


def _sc_pool(x2, table_p):
    mesh = plsc.VectorSubcoreMesh(
        core_axis_name="c", subcore_axis_name="s",
        num_cores=_NC, num_subcores=_NS)
    return pl.kernel(
        _pool_body,
        out_type=jax.ShapeDtypeStruct((_B, _EP), jnp.float32),
        mesh=mesh,
        compiler_params=pltpu.CompilerParams(use_tc_tiling_on_sc=True),
        scratch_types=[
            pltpu.VMEM((_CPW, _IPC), jnp.int32),
            pltpu.VMEM((_IPC, _EP), jnp.float32),
            pltpu.VMEM((_IPC, _EP), jnp.float32),
            pltpu.VMEM((_IPC, _EP), jnp.float32),
            pltpu.VMEM((_IPC, _EP), jnp.float32),
            pltpu.VMEM((_BPW, _EP), jnp.float32),
            pltpu.SemaphoreType.DMA,
            pltpu.SemaphoreType.DMA,
            pltpu.SemaphoreType.DMA,
            pltpu.SemaphoreType.DMA,
        ],
    )(x2, table_p)


_TPB = 12800


def _tp_body(t_ref, o_ref):
    blk = t_ref[...]
    o_ref[...] = jnp.concatenate(
        [blk.T, jnp.zeros((_TPB, _EP - _E), jnp.float32)], axis=1)


def _tc_pad_transpose(t100):
    return pl.pallas_call(
        _tp_body,
        grid=(pl.cdiv(_V, _TPB),),
        in_specs=[pl.BlockSpec((_E, _TPB), lambda i: (0, i))],
        out_specs=pl.BlockSpec((_TPB, _EP), lambda i: (i, 0)),
        out_shape=jax.ShapeDtypeStruct((_V, _EP), jnp.float32),
    )(t100)


def _mlp_body(p_ref, w1_ref, b1_ref, w2_ref, b2_ref, w3_ref, b3_ref, o_ref):
    h = jnp.dot(p_ref[...], w1_ref[...].T, preferred_element_type=jnp.float32)
    h = jnp.maximum(h + b1_ref[...], 0.0)
    h = jnp.dot(h, w2_ref[...].T, preferred_element_type=jnp.float32)
    h = jnp.maximum(h + b2_ref[...], 0.0)
    o_ref[...] = (
        jnp.dot(h, w3_ref[...].T, preferred_element_type=jnp.float32)
        + b3_ref[...])


def _tc_mlp(pooled, W1p, b1, W2, b2, W3, b3):
    return pl.pallas_call(
        _mlp_body,
        out_shape=jax.ShapeDtypeStruct((_B, W3.shape[0]), jnp.float32),
    )(pooled, W1p, b1.reshape(1, -1), W2, b2.reshape(1, -1),
      W3, b3.reshape(1, -1))


@jax.jit
def kernel(x, table, W1, b1, W2, b2, W3, b3):
    x2 = x.reshape(_B * _L // _IPC, _IPC)
    table_p = _tc_pad_transpose(table.T)
    W1p = jnp.pad(W1, ((0, 0), (0, _EP - _E)))
    pooled = _sc_pool(x2, table_p)
    return _tc_mlp(pooled, W1p, b1, W2, b2, W3, b3)


# speedup vs baseline: 1.2484x; 1.0100x over previous
import jax
import jax.numpy as jnp
from jax import lax
from jax.experimental import pallas as pl
from jax.experimental.pallas import tpu as pltpu
from jax.experimental.pallas import tpu_sc as plsc

_V, _E, _B, _L = 100000, 100, 4096, 200
_EP = 128
_NC, _NS = 2, 16
_NW = _NC * _NS
_BPW = _B // _NW
_IPC = 100
_CPW = _BPW * _L // _IPC
_NBUF = 6


def _pool_body(x_hbm, table_hbm, out_hbm, xbuf, g0, g1, g2, g3, g4, g5,
               accbuf, s0, s1, s2, s3, s4, s5):
    gb = (g0, g1, g2, g3, g4, g5)
    sm = (s0, s1, s2, s3, s4, s5)
    wid = lax.axis_index("s") * _NC + lax.axis_index("c")
    pltpu.sync_copy(x_hbm.at[pl.ds(wid * _CPW, _CPW)], xbuf)

    for p in range(_NBUF):
        pltpu.async_copy(table_hbm.at[xbuf.at[p]], gb[p], sm[p])

    zero = jnp.zeros((16,), jnp.float32)

    def accum(g, acc):
        def r_body(r, acc):
            return tuple(acc[s] + g[r, pl.ds(s * 16, 16)] for s in range(8))
        return lax.fori_loop(0, _IPC, r_body, acc)

    def process(c, n_elems, fire):
        for e in range(n_elems):
            acc = (zero,) * 8
            for h in range(2):
                p = 2 * e + h
                chunk = c + p
                pltpu.make_async_copy(
                    table_hbm.at[xbuf.at[chunk]], gb[p], sm[p]).wait()
                acc = accum(gb[p], acc)
                if fire[p]:
                    pltpu.async_copy(
                        table_hbm.at[xbuf.at[chunk + _NBUF]], gb[p], sm[p])
            elem = (c // 2) + e
            for s in range(8):
                accbuf[elem, pl.ds(s * 16, 16)] = acc[s]

    @pl.loop(0, _CPW - 16, step=_NBUF)
    def _main(c):
        process(c, 3, (True,) * 6)

    process(_CPW - 16, 3, (True,) * 6)
    process(_CPW - 10, 3, (True, True, True, True, False, False))
    process(_CPW - 4, 2, (False,) * 6)

    pltpu.sync_copy(accbuf, out_hbm.at[pl.ds(wid * _BPW, _BPW)])


def _sc_pool(x2, table_p):
    mesh = plsc.VectorSubcoreMesh(
        core_axis_name="c", subcore_axis_name="s",
        num_cores=_NC, num_subcores=_NS)
    return pl.kernel(
        _pool_body,
        out_type=jax.ShapeDtypeStruct((_B, _EP), jnp.float32),
        mesh=mesh,
        compiler_params=pltpu.CompilerParams(use_tc_tiling_on_sc=True),
        scratch_types=(
            [pltpu.VMEM((_CPW, _IPC), jnp.int32)]
            + [pltpu.VMEM((_IPC, _EP), jnp.float32)] * _NBUF
            + [pltpu.VMEM((_BPW, _EP), jnp.float32)]
            + [pltpu.SemaphoreType.DMA] * _NBUF
        ),
    )(x2, table_p)


_TPB = 12800


def _tp_body(t_ref, o_ref):
    blk = t_ref[...]
    o_ref[...] = jnp.concatenate(
        [blk.T, jnp.zeros((_TPB, _EP - _E), jnp.float32)], axis=1)


def _tc_pad_transpose(t100):
    return pl.pallas_call(
        _tp_body,
        grid=(pl.cdiv(_V, _TPB),),
        in_specs=[pl.BlockSpec((_E, _TPB), lambda i: (0, i))],
        out_specs=pl.BlockSpec((_TPB, _EP), lambda i: (i, 0)),
        out_shape=jax.ShapeDtypeStruct((_V, _EP), jnp.float32),
    )(t100)


def _mlp_body(p_ref, w1_ref, b1_ref, w2_ref, b2_ref, w3_ref, b3_ref, o_ref):
    h = jnp.dot(p_ref[...], w1_ref[...].T, preferred_element_type=jnp.float32)
    h = jnp.maximum(h + b1_ref[...], 0.0)
    h = jnp.dot(h, w2_ref[...].T, preferred_element_type=jnp.float32)
    h = jnp.maximum(h + b2_ref[...], 0.0)
    o_ref[...] = (
        jnp.dot(h, w3_ref[...].T, preferred_element_type=jnp.float32)
        + b3_ref[...])


def _tc_mlp(pooled, W1p, b1, W2, b2, W3, b3):
    return pl.pallas_call(
        _mlp_body,
        out_shape=jax.ShapeDtypeStruct((_B, W3.shape[0]), jnp.float32),
    )(pooled, W1p, b1.reshape(1, -1), W2, b2.reshape(1, -1),
      W3, b3.reshape(1, -1))


@jax.jit
def kernel(x, table, W1, b1, W2, b2, W3, b3):
    x2 = x.reshape(_B * _L // _IPC, _IPC)
    table_p = _tc_pad_transpose(table.T)
    W1p = jnp.pad(W1, ((0, 0), (0, _EP - _E)))
    pooled = _sc_pool(x2, table_p)
    return _tc_mlp(pooled, W1p, b1, W2, b2, W3, b3)


# speedup vs baseline: 1.2517x; 1.0027x over previous
import jax
import jax.numpy as jnp
from jax import lax
from jax.experimental import pallas as pl
from jax.experimental.pallas import tpu as pltpu
from jax.experimental.pallas import tpu_sc as plsc

_V, _E, _B, _L = 100000, 100, 4096, 200
_EP = 128
_NC, _NS = 2, 16
_NW = _NC * _NS
_BPW = _B // _NW
_IPC = 100
_CPW = _BPW * _L // _IPC
_NBUF = 6


def _pool_body(x_hbm, table_hbm, out_hbm, xbuf, g0, g1, g2, g3, g4, g5,
               accbuf, s0, s1, s2, s3, s4, s5):
    gb = (g0, g1, g2, g3, g4, g5)
    sm = (s0, s1, s2, s3, s4, s5)
    wid = lax.axis_index("s") * _NC + lax.axis_index("c")
    pltpu.sync_copy(x_hbm.at[pl.ds(wid * _CPW, _CPW)], xbuf)

    def idx_ref(chunk):
        return xbuf.at[chunk]

    for p in range(_NBUF):
        pltpu.async_copy(table_hbm.at[idx_ref(p)], gb[p], sm[p])

    zero = jnp.zeros((16,), jnp.float32)

    def accum(g, acc):
        def r_body(r, acc):
            return tuple(acc[s] + g[r, pl.ds(s * 16, 16)] for s in range(8))
        return lax.fori_loop(0, _IPC, r_body, acc)

    def process(c, n_elems, fire):
        for e in range(n_elems):
            acc = (zero,) * 8
            for h in range(2):
                p = 2 * e + h
                chunk = c + p
                pltpu.make_async_copy(
                    table_hbm.at[idx_ref(chunk)], gb[p], sm[p]).wait()
                acc = accum(gb[p], acc)
                if fire[p]:
                    pltpu.async_copy(
                        table_hbm.at[idx_ref(chunk + _NBUF)], gb[p], sm[p])
            elem = (c // 2) + e
            for s in range(8):
                accbuf[elem, pl.ds(s * 16, 16)] = acc[s]

    @pl.loop(0, _CPW - 16, step=_NBUF)
    def _main(c):
        process(c, 3, (True,) * 6)

    process(_CPW - 16, 3, (True,) * 6)
    process(_CPW - 10, 3, (True, True, True, True, False, False))
    process(_CPW - 4, 2, (False,) * 6)

    pltpu.sync_copy(accbuf, out_hbm.at[pl.ds(wid * _BPW, _BPW)])


def _sc_pool(x2, table_p):
    mesh = plsc.VectorSubcoreMesh(
        core_axis_name="c", subcore_axis_name="s",
        num_cores=_NC, num_subcores=_NS)
    return pl.kernel(
        _pool_body,
        out_type=jax.ShapeDtypeStruct((_B, _EP), jnp.float32),
        mesh=mesh,
        compiler_params=pltpu.CompilerParams(use_tc_tiling_on_sc=True),
        scratch_types=(
            [pltpu.VMEM((_CPW, _IPC), jnp.int32)]
            + [pltpu.VMEM((_IPC, _EP), jnp.float32)] * _NBUF
            + [pltpu.VMEM((_BPW, _EP), jnp.float32)]
            + [pltpu.SemaphoreType.DMA] * _NBUF
        ),
    )(x2, table_p)


_TPB = 25600


def _tp_body(t_ref, o_ref):
    blk = t_ref[...]
    o_ref[...] = jnp.concatenate(
        [blk.T, jnp.zeros((_TPB, _EP - _E), jnp.float32)], axis=1)


def _tc_pad_transpose(t100):
    return pl.pallas_call(
        _tp_body,
        grid=(pl.cdiv(_V, _TPB),),
        in_specs=[pl.BlockSpec((_E, _TPB), lambda i: (0, i))],
        out_specs=pl.BlockSpec((_TPB, _EP), lambda i: (i, 0)),
        out_shape=jax.ShapeDtypeStruct((_V, _EP), jnp.float32),
    )(t100)


def _mlp_body(p_ref, w1_ref, b1_ref, w2_ref, b2_ref, w3_ref, b3_ref, o_ref):
    h = jnp.dot(p_ref[...], w1_ref[...].T, preferred_element_type=jnp.float32)
    h = jnp.maximum(h + b1_ref[...], 0.0)
    h = jnp.dot(h, w2_ref[...].T, preferred_element_type=jnp.float32)
    h = jnp.maximum(h + b2_ref[...], 0.0)
    o_ref[...] = (
        jnp.dot(h, w3_ref[...].T, preferred_element_type=jnp.float32)
        + b3_ref[...])


def _tc_mlp(pooled, W1p, b1, W2, b2, W3, b3):
    return pl.pallas_call(
        _mlp_body,
        out_shape=jax.ShapeDtypeStruct((_B, W3.shape[0]), jnp.float32),
    )(pooled, W1p, b1.reshape(1, -1), W2, b2.reshape(1, -1),
      W3, b3.reshape(1, -1))


@jax.jit
def kernel(x, table, W1, b1, W2, b2, W3, b3):
    x2 = x.reshape(_B * _L // _IPC, _IPC)
    table_p = _tc_pad_transpose(table.T)
    W1p = jnp.pad(W1, ((0, 0), (0, _EP - _E)))
    pooled = _sc_pool(x2, table_p)
    return _tc_mlp(pooled, W1p, b1, W2, b2, W3, b3)
